# Initial kernel scaffold; baseline (speedup 1.0000x reference)
import jax
import jax.numpy as jnp
from jax.experimental import pallas as pl


def kernel(x, patterns):
    raise NotImplementedError("write your pallas kernel here")


# speedup vs baseline: 1.8309x; 1.8309x over previous
import jax
import jax.numpy as jnp
from jax.experimental import pallas as pl

_W = 20
_H = 20
_WH = _W * _H
_S = 61
_PC = 145
_MO = 8
_NT = 8 * _MO
_CT = _PC + 1 + 1 + _NT


def _label_kernel(x_ref, lab_ref):
    x = x_ref[...]
    fg = x != 0
    row = jax.lax.broadcasted_iota(jnp.int32, (_W, _H), 0)
    col = jax.lax.broadcasted_iota(jnp.int32, (_W, _H), 1)
    bg = jnp.int32(_WH)
    lab0 = jnp.where(fg, row * _H + col, bg)
    colpad = jnp.full((_W, 1), bg, jnp.int32)
    rowpad = jnp.full((1, _H), bg, jnp.int32)

    def body(i, lab):
        dn = jnp.concatenate([lab[1:, :], rowpad], axis=0)
        up = jnp.concatenate([rowpad, lab[:-1, :]], axis=0)
        rt = jnp.concatenate([lab[:, 1:], colpad], axis=1)
        lf = jnp.concatenate([colpad, lab[:, :-1]], axis=1)
        nb = jnp.minimum(jnp.minimum(up, dn), jnp.minimum(lf, rt))
        return jnp.where(fg, jnp.minimum(lab, nb), bg)

    lab_ref[...] = jax.lax.fori_loop(0, _WH, body, lab0)


def _main_kernel(p2_ref, xa_ref, xb_ref, cma_ref, cmb_ref, cpt_ref, out_ref):
    xi = pl.program_id(0)
    xj = pl.program_id(1)

    slabf = p2_ref[0]
    base = (xj - xi) * _H
    for yi in range(_W):
        out_ref[0, yi, 0, :, 0:_PC] = slabf[30 - yi:50 - yi, :]
        s = base - yi
        s = jax.lax.rem(s, _WH + 1)
        s = jnp.where(s < 0, s + _WH + 1, s)
        out_ref[0, yi, 0, :, _PC + 2:_CT] = cpt_ref[pl.ds(s, _H), :]

    xa = xa_ref[0]
    xb = xb_ref[0]
    eq = (xa.T == xb).astype(jnp.float32)
    out_ref[0, :, 0, :, _PC:_PC + 1] = eq.reshape(_W, _H, 1)

    cma = cma_ref[0]
    cmb = cmb_ref[0]
    ov = jax.lax.dot_general(cma, cmb, (((0,), (0,)), ((), ())),
                             preferred_element_type=jnp.float32)
    out_ref[0, :, 0, :, _PC + 1:_PC + 2] = ov.reshape(_W, _H, 1)


def kernel(x, patterns):
    if x.ndim == 3:
        x = x[0]
    x = x.astype(jnp.int32)
    patterns = patterns.astype(jnp.float32)

    lab = pl.pallas_call(
        _label_kernel,
        out_shape=jax.ShapeDtypeStruct((_W, _H), jnp.int32),
    )(x)

    counts = jnp.bincount(lab.reshape(-1), length=_WH + 1)
    sizes = counts[:_WH].astype(jnp.int32)
    labels = jnp.arange(_WH, dtype=jnp.int32)
    key = jnp.where(sizes > 1, (_WH + 1 - sizes) * _WH + labels,
                    (_WH + 2) * _WH + labels)
    order = jnp.argsort(key)
    sel = order[:_MO]
    valid = sizes[sel] > 1
    masks = ((lab[None, :, :] == sel[:, None, None])
             & valid[:, None, None]).astype(jnp.float32)

    def topleft(m):
        r0 = jnp.argmax(jnp.any(m > 0, axis=1))
        c0 = jnp.argmax(jnp.any(m > 0, axis=0))
        return jnp.roll(m, (-r0, -c0), axis=(0, 1))

    def transforms(m):
        t = m.T
        vs = [m, m[::-1, :], m[:, ::-1], m[::-1, ::-1],
              t, t[::-1, :], t[:, ::-1], t[::-1, ::-1]]
        return jnp.stack([topleft(v) for v in vs], axis=0)

    c = jax.vmap(transforms)(masks).reshape(_NT, _WH)
    cpad = jnp.pad(c, ((0, 0), (0, 1)))
    cpt = cpad.T
    cpt2 = jnp.concatenate([cpt, cpt], axis=0)

    p2 = patterns.reshape(_S, _S, _PC)[:, ::-1, :]
    x3 = x.reshape(_W, 1, _H)
    cmr = masks.reshape(_MO, _W, _H).transpose(1, 0, 2)

    out5 = pl.pallas_call(
        _main_kernel,
        grid=(_W, _W),
        in_specs=[
            pl.BlockSpec((1, _S, _PC), lambda i, j: (i - j + 30, 0, 0)),
            pl.BlockSpec((1, 1, _H), lambda i, j: (i, 0, 0)),
            pl.BlockSpec((1, 1, _H), lambda i, j: (j, 0, 0)),
            pl.BlockSpec((1, _MO, _H), lambda i, j: (i, 0, 0)),
            pl.BlockSpec((1, _MO, _H), lambda i, j: (j, 0, 0)),
            pl.BlockSpec((2 * (_WH + 1), _NT), lambda i, j: (0, 0)),
        ],
        out_specs=pl.BlockSpec((1, _H, 1, _H, _CT),
                               lambda i, j: (i, 0, j, 0, 0)),
        out_shape=jax.ShapeDtypeStruct((_W, _H, _W, _H, _CT), jnp.float32),
    )(p2, x3, x3, cmr, cmr, cpt2)

    return out5.reshape(_WH, _WH, _CT)
